# baseline (speedup 1.0000x reference)
import functools

import jax
import jax.numpy as jnp
from jax import lax
from jax.experimental import pallas as pl
from jax.experimental.pallas import tpu as pltpu
from jax.experimental.pallas import tpu_sc as plsc

_N = 4096
_K = 16
_L = 16
_NCH = _N // _L
_R = 8
_NW = 32
_DEPTH = _N // _L


def _splat(ref, gi):
    return plsc.load_gather(ref, [gi])


def _sc_body(xr_h, yr_h, zr_h, xp_h, yp_h, zp_h, out_h,
             vxr, vyr, vzr, vsq, vxp, vyp, vzp,
             vda, vdb, vdc, vdd, vde, vdf, vdg, vdh,
             vcand, vcol, vacc,
             *, nbatch, rows_per_w):
    tecs_per_b = _NW // nbatch
    cid = lax.axis_index("c")
    sid = lax.axis_index("s")
    wid = sid * 2 + cid
    bat = wid // tecs_per_b
    row0 = (wid % tecs_per_b) * rows_per_w

    pltpu.sync_copy(xr_h.at[bat], vxr)
    pltpu.sync_copy(yr_h.at[bat], vyr)
    pltpu.sync_copy(zr_h.at[bat], vzr)
    pltpu.sync_copy(xp_h.at[bat], vxp)
    pltpu.sync_copy(yp_h.at[bat], vyp)
    pltpu.sync_copy(zp_h.at[bat], vzp)

    inf_v = jnp.full((_L,), jnp.inf, jnp.float32)
    iot = lax.iota(jnp.int32, _L)

    @pl.loop(0, _N, step=_L)
    def _(o):
        x = vxr[pl.ds(o, _L)]
        y = vyr[pl.ds(o, _L)]
        z = vzr[pl.ds(o, _L)]
        vsq[pl.ds(o, _L)] = x * x + y * y + z * z
        vcol[pl.ds(o, _L)] = jnp.broadcast_to(o, (_L,)) + iot
        vcand[pl.ds(o, _L)] = jnp.zeros((_L,), jnp.int32)

    drefs = (vda, vdb, vdc, vdd, vde, vdf, vdg, vdh)
    lane_base = iot * _DEPTH

    def group_body(g, acc):
        rb = row0 + g * _R
        giv = [jnp.broadcast_to(rb + r, (_L,)) for r in range(_R)]
        axv, ayv, azv = [], [], []
        for r in range(_R):
            axv.append(-2.0 * _splat(vxr, giv[r]))
            ayv.append(-2.0 * _splat(vyr, giv[r]))
            azv.append(-2.0 * _splat(vzr, giv[r]))

        def sweep(c, mv):
            base = c * _L
            xs = vxr[pl.ds(base, _L)]
            ys = vyr[pl.ds(base, _L)]
            zs = vzr[pl.ds(base, _L)]
            qs = vsq[pl.ds(base, _L)]
            colv = vcol[pl.ds(base, _L)]
            out = []
            for r in range(_R):
                t = qs + xs * axv[r]
                t = t + ys * ayv[r]
                t = t + zs * azv[r]
                t = jnp.where(colv == giv[r], jnp.inf, t)
                drefs[r][pl.ds(base, _L)] = t
                out.append(jnp.minimum(mv[r], t))
            return tuple(out)

        mvs = plsc.parallel_loop(0, _NCH, unroll=2,
                                 carry=(inf_v,) * _R)(sweep)

        for r in range(_R):
            dr = drefs[r]
            tauv = jnp.broadcast_to(jnp.max(mvs[r]), (_L,))

            def collect(c, cntv):
                base = c * _L
                dv = dr[pl.ds(base, _L)]
                msk = dv <= tauv
                colv = vcol[pl.ds(base, _L)]
                plsc.store_scatter(vcand, [lane_base + cntv], colv,
                                   mask=msk)
                return cntv + jnp.where(msk, 1, 0)

            cntv = plsc.parallel_loop(
                0, _NCH, unroll=4,
                carry=jnp.zeros((_L,), jnp.int32))(collect)
            maxc = jnp.max(cntv)

            def merge(t, c):
                tk, ti = c
                tv = jnp.broadcast_to(t, (_L,))
                idxc = plsc.load_gather(vcand, [lane_base + tv])
                dc = plsc.load_gather(dr, [idxc])
                dc = jnp.where(tv < cntv, dc, jnp.inf)
                sk, si = plsc.sort_key_val(dc, idxc)
                rk = jnp.flip(sk)
                ri = jnp.flip(si)
                keep = tk <= rk
                lk = jnp.where(keep, tk, rk)
                li = jnp.where(keep, ti, ri)
                return tuple(plsc.sort_key_val(lk, li))

            tk, ti = lax.fori_loop(0, maxc, merge, (inf_v, giv[r]),
                                   unroll=False)

            xig = _splat(vxr, giv[r])
            yig = _splat(vyr, giv[r])
            zig = _splat(vzr, giv[r])
            dref = tk + xig * xig + yig * yig + zig * zig
            px = _splat(vxp, ti) - _splat(vxp, giv[r])
            py = _splat(vyp, ti) - _splat(vyp, giv[r])
            pz = _splat(vzp, ti) - _splat(vzp, giv[r])
            dp = px * px + py * py + pz * pz
            q = dp / dref
            qi = lax.bitcast_convert_type(q, jnp.int32)
            s = lax.bitcast_convert_type(
                jnp.full((_L,), 0x1FBD1DF5, jnp.int32)
                + lax.shift_right_logical(qi, 1), jnp.float32)
            for _ in range(3):
                s = 0.5 * (s + q / s)
            acc = acc + jnp.maximum(s - 1.0, 0.0)
        return acc

    acc = lax.fori_loop(0, rows_per_w // _R, group_body,
                        jnp.zeros((_L,), jnp.float32), unroll=False)
    vacc[...] = acc
    pltpu.sync_copy(vacc, out_h.at[wid])


def kernel(points_ref, points):
    nbatch, n, _ = points_ref.shape
    rows_per_w = nbatch * n // _NW
    mesh = plsc.VectorSubcoreMesh(core_axis_name="c", subcore_axis_name="s")
    body = functools.partial(_sc_body, nbatch=nbatch, rows_per_w=rows_per_w)
    run = pl.kernel(
        body,
        out_type=jax.ShapeDtypeStruct((_NW, _L), jnp.float32),
        mesh=mesh,
        compiler_params=pltpu.CompilerParams(needs_layout_passes=False),
        scratch_types=[
            pltpu.VMEM((n,), jnp.float32),
            pltpu.VMEM((n,), jnp.float32),
            pltpu.VMEM((n,), jnp.float32),
            pltpu.VMEM((n,), jnp.float32),
            pltpu.VMEM((n,), jnp.float32),
            pltpu.VMEM((n,), jnp.float32),
            pltpu.VMEM((n,), jnp.float32),
            pltpu.VMEM((n,), jnp.float32),
            pltpu.VMEM((n,), jnp.float32),
            pltpu.VMEM((n,), jnp.float32),
            pltpu.VMEM((n,), jnp.float32),
            pltpu.VMEM((n,), jnp.float32),
            pltpu.VMEM((n,), jnp.float32),
            pltpu.VMEM((n,), jnp.float32),
            pltpu.VMEM((n,), jnp.float32),
            pltpu.VMEM((n,), jnp.int32),
            pltpu.VMEM((n,), jnp.int32),
            pltpu.VMEM((_L,), jnp.float32),
        ],
    )
    pr = jnp.transpose(points_ref, (0, 2, 1))
    pp = jnp.transpose(points, (0, 2, 1))
    out = run(pr[:, 0], pr[:, 1], pr[:, 2], pp[:, 0], pp[:, 1], pp[:, 2])
    return jnp.sum(out) / jnp.float32(nbatch * n * _K)


# speedup vs baseline: 1.1848x; 1.1848x over previous
import functools

import jax
import jax.numpy as jnp
from jax import lax
from jax.experimental import pallas as pl
from jax.experimental.pallas import tpu as pltpu
from jax.experimental.pallas import tpu_sc as plsc

_N = 4096
_K = 16
_L = 16
_NCH = _N // _L
_R = 4
_NW = 32
_DEPTH = _N // _L


def _splat(ref, gi):
    return plsc.load_gather(ref, [gi])


def _sc_body(xr_h, yr_h, zr_h, xp_h, yp_h, zp_h, out_h,
             vxr, vyr, vzr, vsq, vxp, vyp, vzp,
             vda, vdb, vdc, vdd, vcand, vcol, vacc,
             *, nbatch, rows_per_w):
    tecs_per_b = _NW // nbatch
    cid = lax.axis_index("c")
    sid = lax.axis_index("s")
    wid = sid * 2 + cid
    bat = wid // tecs_per_b
    row0 = (wid % tecs_per_b) * rows_per_w

    pltpu.sync_copy(xr_h.at[bat], vxr)
    pltpu.sync_copy(yr_h.at[bat], vyr)
    pltpu.sync_copy(zr_h.at[bat], vzr)
    pltpu.sync_copy(xp_h.at[bat], vxp)
    pltpu.sync_copy(yp_h.at[bat], vyp)
    pltpu.sync_copy(zp_h.at[bat], vzp)

    inf_v = jnp.full((_L,), jnp.inf, jnp.float32)
    iot = lax.iota(jnp.int32, _L)

    @pl.loop(0, _N, step=_L)
    def _(o):
        x = vxr[pl.ds(o, _L)]
        y = vyr[pl.ds(o, _L)]
        z = vzr[pl.ds(o, _L)]
        vsq[pl.ds(o, _L)] = x * x + y * y + z * z
        vcol[pl.ds(o, _L)] = jnp.broadcast_to(o, (_L,)) + iot
        vcand[pl.ds(o, _L)] = jnp.zeros((_L,), jnp.int32)

    drefs = (vda, vdb, vdc, vdd)
    lane_base = iot * _DEPTH

    def group_body(g, acc):
        rb = row0 + g * _R
        giv = [jnp.broadcast_to(rb + r, (_L,)) for r in range(_R)]
        axv, ayv, azv = [], [], []
        for r in range(_R):
            axv.append(-2.0 * _splat(vxr, giv[r]))
            ayv.append(-2.0 * _splat(vyr, giv[r]))
            azv.append(-2.0 * _splat(vzr, giv[r]))

        def sweep(c, mv):
            base = c * _L
            xs = vxr[pl.ds(base, _L)]
            ys = vyr[pl.ds(base, _L)]
            zs = vzr[pl.ds(base, _L)]
            qs = vsq[pl.ds(base, _L)]
            colv = vcol[pl.ds(base, _L)]
            out = []
            for r in range(_R):
                t = qs + xs * axv[r]
                t = t + ys * ayv[r]
                t = t + zs * azv[r]
                t = jnp.where(colv == giv[r], jnp.inf, t)
                drefs[r][pl.ds(base, _L)] = t
                out.append(jnp.minimum(mv[r], t))
            return tuple(out)

        mvs = plsc.parallel_loop(0, _NCH, unroll=4,
                                 carry=(inf_v,) * _R)(sweep)

        for r in range(_R):
            dr = drefs[r]
            tauv = jnp.broadcast_to(jnp.max(mvs[r]), (_L,))

            def collect(c, cntv):
                base = c * _L
                dv = dr[pl.ds(base, _L)]
                msk = dv <= tauv
                colv = vcol[pl.ds(base, _L)]
                plsc.store_scatter(vcand, [lane_base + cntv], colv,
                                   mask=msk)
                return cntv + jnp.where(msk, 1, 0)

            cntv = plsc.parallel_loop(
                0, _NCH, unroll=8,
                carry=jnp.zeros((_L,), jnp.int32))(collect)
            maxc = jnp.max(cntv)

            def merge(t, c):
                tk, ti = c
                tv = jnp.broadcast_to(t, (_L,))
                idxc = plsc.load_gather(vcand, [lane_base + tv])
                dc = plsc.load_gather(dr, [idxc])
                dc = jnp.where(tv < cntv, dc, jnp.inf)
                sk, si = plsc.sort_key_val(dc, idxc)
                rk = jnp.flip(sk)
                ri = jnp.flip(si)
                keep = tk <= rk
                lk = jnp.where(keep, tk, rk)
                li = jnp.where(keep, ti, ri)
                return tuple(plsc.sort_key_val(lk, li))

            tk, ti = lax.fori_loop(0, maxc, merge, (inf_v, giv[r]),
                                   unroll=False)

            xig = _splat(vxr, giv[r])
            yig = _splat(vyr, giv[r])
            zig = _splat(vzr, giv[r])
            dref = tk + xig * xig + yig * yig + zig * zig
            px = _splat(vxp, ti) - _splat(vxp, giv[r])
            py = _splat(vyp, ti) - _splat(vyp, giv[r])
            pz = _splat(vzp, ti) - _splat(vzp, giv[r])
            dp = px * px + py * py + pz * pz
            q = dp / dref
            qi = lax.bitcast_convert_type(q, jnp.int32)
            s = lax.bitcast_convert_type(
                jnp.full((_L,), 0x1FBD1DF5, jnp.int32)
                + lax.shift_right_logical(qi, 1), jnp.float32)
            for _ in range(3):
                s = 0.5 * (s + q / s)
            acc = acc + jnp.maximum(s - 1.0, 0.0)
        return acc

    acc = lax.fori_loop(0, rows_per_w // _R, group_body,
                        jnp.zeros((_L,), jnp.float32), unroll=False)
    vacc[...] = acc
    pltpu.sync_copy(vacc, out_h.at[wid])


def kernel(points_ref, points):
    nbatch, n, _ = points_ref.shape
    rows_per_w = nbatch * n // _NW
    mesh = plsc.VectorSubcoreMesh(core_axis_name="c", subcore_axis_name="s")
    body = functools.partial(_sc_body, nbatch=nbatch, rows_per_w=rows_per_w)
    run = pl.kernel(
        body,
        out_type=jax.ShapeDtypeStruct((_NW, _L), jnp.float32),
        mesh=mesh,
        compiler_params=pltpu.CompilerParams(needs_layout_passes=False),
        scratch_types=[
            pltpu.VMEM((n,), jnp.float32),
            pltpu.VMEM((n,), jnp.float32),
            pltpu.VMEM((n,), jnp.float32),
            pltpu.VMEM((n,), jnp.float32),
            pltpu.VMEM((n,), jnp.float32),
            pltpu.VMEM((n,), jnp.float32),
            pltpu.VMEM((n,), jnp.float32),
            pltpu.VMEM((n,), jnp.float32),
            pltpu.VMEM((n,), jnp.float32),
            pltpu.VMEM((n,), jnp.float32),
            pltpu.VMEM((n,), jnp.float32),
            pltpu.VMEM((n,), jnp.int32),
            pltpu.VMEM((n,), jnp.int32),
            pltpu.VMEM((_L,), jnp.float32),
        ],
    )
    pr = jnp.transpose(points_ref, (0, 2, 1))
    pp = jnp.transpose(points, (0, 2, 1))
    out = run(pr[:, 0], pr[:, 1], pr[:, 2], pp[:, 0], pp[:, 1], pp[:, 2])
    return jnp.sum(out) / jnp.float32(nbatch * n * _K)
